# Initial kernel scaffold; baseline (speedup 1.0000x reference)
import jax
import jax.numpy as jnp
from jax.experimental import pallas as pl


def kernel(data_x, edge_index, W_rel, b_rel, W_root, W_hist, b_hist, W_ls, b_ls, W1, b1, W2, b2):
    raise NotImplementedError("write your pallas kernel here")


# speedup vs baseline: 22.8487x; 22.8487x over previous
import functools

import jax
import jax.numpy as jnp
from jax import lax
from jax.experimental import pallas as pl
from jax.experimental.pallas import tpu as pltpu
from jax.experimental.pallas import tpu_sc as plsc

N = 10000
E = 320000
IN_DIM = 128
HIDDEN = 256
OUT_DIM = 128

NC = 2
NS = 16
NW = NC * NS
CHUNK = 128
ROWS_PER_W = 80
E_PAD = NW * ROWS_PER_W * CHUNK
ACC_PAD = 10240
TILE_SLICE = ACC_PAD // NS
ROW_TILE = 1024


def _sc_segment_sum(s, src3, dst3):
    mesh = plsc.VectorSubcoreMesh(core_axis_name="c", subcore_axis_name="s")

    @functools.partial(
        pl.kernel,
        mesh=mesh,
        out_type=jax.ShapeDtypeStruct((NC, ACC_PAD), jnp.float32),
        scratch_types=[
            pltpu.VMEM((ROWS_PER_W, CHUNK), jnp.int32),
            pltpu.VMEM((ROWS_PER_W, CHUNK), jnp.int32),
            pltpu.VMEM((ROWS_PER_W, CHUNK), jnp.float32),
            pltpu.VMEM((TILE_SLICE,), jnp.float32),
            pltpu.VMEM_SHARED((ACC_PAD,), jnp.float32),
            pltpu.SemaphoreType.DMA,
            pltpu.SemaphoreType.DMA,
        ],
    )
    def seg_sum(s_hbm, src_hbm, dst_hbm, out_hbm, src_v, dst_v, vals_v,
                z_v, acc, sem_g, sem_s):
        cid = lax.axis_index("c")
        sid = lax.axis_index("s")
        wid = cid * NS + sid

        pltpu.sync_copy(src_hbm.at[wid], src_v)
        pltpu.sync_copy(dst_hbm.at[wid], dst_v)

        def gfire(j, carry):
            pltpu.async_copy(s_hbm.at[src_v.at[j]], vals_v.at[j], sem_g)
            return carry

        lax.fori_loop(0, ROWS_PER_W, gfire, 0)

        zeros16 = jnp.zeros((16,), jnp.float32)

        def zbody(k, carry):
            z_v[pl.ds(k * 16, 16)] = zeros16
            return carry

        lax.fori_loop(0, TILE_SLICE // 16, zbody, 0)
        pltpu.sync_copy(z_v, acc.at[pl.ds(sid * TILE_SLICE, TILE_SLICE)])
        plsc.subcore_barrier()

        def gdrain(j, carry):
            pltpu.make_async_copy(s_hbm.at[src_v.at[j]], vals_v.at[j],
                                  sem_g).wait()
            return carry

        lax.fori_loop(0, ROWS_PER_W, gdrain, 0)

        def sfire(j, carry):
            pltpu.async_copy(vals_v.at[j], acc.at[dst_v.at[j]], sem_s,
                             add=True)
            return carry

        lax.fori_loop(0, ROWS_PER_W, sfire, 0)

        def sdrain(j, carry):
            pltpu.make_async_copy(vals_v.at[j], acc.at[dst_v.at[j]],
                                  sem_s).wait()
            return carry

        lax.fori_loop(0, ROWS_PER_W, sdrain, 0)
        plsc.subcore_barrier()

        pltpu.sync_copy(
            acc.at[pl.ds(sid * TILE_SLICE, TILE_SLICE)],
            out_hbm.at[cid, pl.ds(sid * TILE_SLICE, TILE_SLICE)],
        )

    return seg_sum(s, src3, dst3)


def _tc_body(x_ref, p_ref, wrel, brel, wroot, wh, bh, wls, bls, w1, b1, w2,
             b2, o_ref):
    xb = x_ref[...]
    pb = p_ref[...]
    ones = jnp.ones((NC, 1), jnp.float32)
    aggc = lax.dot_general(pb, ones, (((0,), (0,)), ((), ())),
                           preferred_element_type=jnp.float32)
    x1 = aggc * wrel[...] + brel[...] + xb[:, 0:1] * wroot[...]
    h = (jnp.dot(xb, wh[...], preferred_element_type=jnp.float32) + bh[...]
         + jnp.dot(x1, wls[...], preferred_element_type=jnp.float32)
         + bls[...])
    h1 = jnp.maximum(
        jnp.dot(h, w1[...], preferred_element_type=jnp.float32) + b1[...], 0.0)
    o_ref[...] = jax.nn.sigmoid(
        jnp.dot(h1, w2[...], preferred_element_type=jnp.float32) + b2[...])


def _tc_forward(data_x, parts, W_rel, b_rel, W_root, W_hist, b_hist, W_ls,
                b_ls, W1, b1, W2, b2):
    full = lambda shape: pl.BlockSpec(shape, lambda i: (0, 0))
    return pl.pallas_call(
        _tc_body,
        grid=(pl.cdiv(N, ROW_TILE),),
        in_specs=[
            pl.BlockSpec((ROW_TILE, IN_DIM), lambda i: (i, 0)),
            pl.BlockSpec((NC, ROW_TILE), lambda i: (0, i)),
            full((1, 1)), full((1, 1)), full((1, 1)),
            full((IN_DIM, HIDDEN)), full((1, HIDDEN)),
            full((1, HIDDEN)), full((1, HIDDEN)),
            full((HIDDEN, HIDDEN)), full((1, HIDDEN)),
            full((HIDDEN, OUT_DIM)), full((1, OUT_DIM)),
        ],
        out_specs=pl.BlockSpec((ROW_TILE, OUT_DIM), lambda i: (i, 0)),
        out_shape=jax.ShapeDtypeStruct((N, OUT_DIM), jnp.float32),
        compiler_params=pltpu.CompilerParams(
            dimension_semantics=("arbitrary",)),
    )(data_x, parts, W_rel, b_rel, W_root, W_hist, b_hist, W_ls, b_ls, W1,
      b1, W2, b2)


@jax.jit
def kernel(data_x, edge_index, W_rel, b_rel, W_root, W_hist, b_hist, W_ls,
           b_ls, W1, b1, W2, b2):
    s = data_x[:, 0]
    pad = E_PAD - E
    src_p = jnp.concatenate([edge_index[0], jnp.zeros((pad,), jnp.int32)])
    dst_p = jnp.concatenate([edge_index[1], jnp.full((pad,), N, jnp.int32)])
    src3 = src_p.reshape(NW, ROWS_PER_W, CHUNK)
    dst3 = dst_p.reshape(NW, ROWS_PER_W, CHUNK)

    parts = _sc_segment_sum(s, src3, dst3)

    return _tc_forward(
        data_x, parts, W_rel, b_rel.reshape(1, 1), W_root, W_hist,
        b_hist.reshape(1, HIDDEN), W_ls, b_ls.reshape(1, HIDDEN), W1,
        b1.reshape(1, HIDDEN), W2, b2.reshape(1, OUT_DIM))


# speedup vs baseline: 37.0371x; 1.6210x over previous
import functools

import jax
import jax.numpy as jnp
from jax import lax
from jax.experimental import pallas as pl
from jax.experimental.pallas import tpu as pltpu
from jax.experimental.pallas import tpu_sc as plsc

N = 10000
E = 320000
IN_DIM = 128
HIDDEN = 256
OUT_DIM = 128

NC = 2
NS = 16
NW = NC * NS
CHUNK = 128
ROWS_PER_W = 80
E_PAD = NW * ROWS_PER_W * CHUNK
ACC_PAD = 10240
TILE_SLICE = ACC_PAD // NS
ROW_TILE = 1024


def _sc_segment_sum(s, src3, dst3):
    mesh = plsc.VectorSubcoreMesh(core_axis_name="c", subcore_axis_name="s")

    @functools.partial(
        pl.kernel,
        mesh=mesh,
        out_type=jax.ShapeDtypeStruct((NC, ACC_PAD), jnp.float32),
        scratch_types=[
            pltpu.VMEM((ROWS_PER_W, CHUNK), jnp.int32),
            pltpu.VMEM((ROWS_PER_W, CHUNK), jnp.int32),
            pltpu.VMEM((ROWS_PER_W, CHUNK), jnp.float32),
            pltpu.VMEM((TILE_SLICE,), jnp.float32),
            pltpu.VMEM_SHARED((ACC_PAD,), jnp.float32),
            pltpu.VMEM_SHARED((N,), jnp.float32),
            pltpu.SemaphoreType.DMA,
            pltpu.SemaphoreType.DMA,
        ],
    )
    def seg_sum(s_hbm, src_hbm, dst_hbm, out_hbm, src_v, dst_v, vals_v,
                z_v, acc, s_sh, sem_g, sem_s):
        cid = lax.axis_index("c")
        sid = lax.axis_index("s")
        wid = cid * NS + sid

        pltpu.sync_copy(src_hbm.at[wid], src_v)
        pltpu.sync_copy(dst_hbm.at[wid], dst_v)

        @pl.when(sid == 0)
        def _():
            pltpu.sync_copy(s_hbm, s_sh)

        zeros16 = jnp.zeros((16,), jnp.float32)

        def zbody(k, carry):
            z_v[pl.ds(k * 16, 16)] = zeros16
            return carry

        lax.fori_loop(0, TILE_SLICE // 16, zbody, 0)
        pltpu.sync_copy(z_v, acc.at[pl.ds(sid * TILE_SLICE, TILE_SLICE)])
        plsc.subcore_barrier()

        def gfire(j, carry):
            pltpu.async_copy(s_sh.at[src_v.at[j]], vals_v.at[j], sem_g)
            return carry

        lax.fori_loop(0, ROWS_PER_W, gfire, 0)

        def gdrain(j, carry):
            pltpu.make_async_copy(s_sh.at[src_v.at[j]], vals_v.at[j],
                                  sem_g).wait()
            return carry

        lax.fori_loop(0, ROWS_PER_W, gdrain, 0)

        def sfire(j, carry):
            pltpu.async_copy(vals_v.at[j], acc.at[dst_v.at[j]], sem_s,
                             add=True)
            return carry

        lax.fori_loop(0, ROWS_PER_W, sfire, 0)

        def sdrain(j, carry):
            pltpu.make_async_copy(vals_v.at[j], acc.at[dst_v.at[j]],
                                  sem_s).wait()
            return carry

        lax.fori_loop(0, ROWS_PER_W, sdrain, 0)
        plsc.subcore_barrier()

        pltpu.sync_copy(
            acc.at[pl.ds(sid * TILE_SLICE, TILE_SLICE)],
            out_hbm.at[cid, pl.ds(sid * TILE_SLICE, TILE_SLICE)],
        )

    return seg_sum(s, src3, dst3)


def _tc_body(x_ref, p_ref, wrel, brel, wroot, wh, bh, wls, bls, w1, b1, w2,
             b2, o_ref):
    xb = x_ref[...]
    pb = p_ref[...]
    ones = jnp.ones((NC, 1), jnp.float32)
    aggc = lax.dot_general(pb, ones, (((0,), (0,)), ((), ())),
                           preferred_element_type=jnp.float32)
    x1 = aggc * wrel[...] + brel[...] + xb[:, 0:1] * wroot[...]
    h = (jnp.dot(xb, wh[...], preferred_element_type=jnp.float32) + bh[...]
         + jnp.dot(x1, wls[...], preferred_element_type=jnp.float32)
         + bls[...])
    h1 = jnp.maximum(
        jnp.dot(h, w1[...], preferred_element_type=jnp.float32) + b1[...], 0.0)
    o_ref[...] = jax.nn.sigmoid(
        jnp.dot(h1, w2[...], preferred_element_type=jnp.float32) + b2[...])


def _tc_forward(data_x, parts, W_rel, b_rel, W_root, W_hist, b_hist, W_ls,
                b_ls, W1, b1, W2, b2):
    full = lambda shape: pl.BlockSpec(shape, lambda i: (0, 0))
    return pl.pallas_call(
        _tc_body,
        grid=(pl.cdiv(N, ROW_TILE),),
        in_specs=[
            pl.BlockSpec((ROW_TILE, IN_DIM), lambda i: (i, 0)),
            pl.BlockSpec((NC, ROW_TILE), lambda i: (0, i)),
            full((1, 1)), full((1, 1)), full((1, 1)),
            full((IN_DIM, HIDDEN)), full((1, HIDDEN)),
            full((1, HIDDEN)), full((1, HIDDEN)),
            full((HIDDEN, HIDDEN)), full((1, HIDDEN)),
            full((HIDDEN, OUT_DIM)), full((1, OUT_DIM)),
        ],
        out_specs=pl.BlockSpec((ROW_TILE, OUT_DIM), lambda i: (i, 0)),
        out_shape=jax.ShapeDtypeStruct((N, OUT_DIM), jnp.float32),
        compiler_params=pltpu.CompilerParams(
            dimension_semantics=("arbitrary",)),
    )(data_x, parts, W_rel, b_rel, W_root, W_hist, b_hist, W_ls, b_ls, W1,
      b1, W2, b2)


@jax.jit
def kernel(data_x, edge_index, W_rel, b_rel, W_root, W_hist, b_hist, W_ls,
           b_ls, W1, b1, W2, b2):
    s = data_x[:, 0]
    pad = E_PAD - E
    src_p = jnp.concatenate([edge_index[0], jnp.zeros((pad,), jnp.int32)])
    dst_p = jnp.concatenate([edge_index[1], jnp.full((pad,), N, jnp.int32)])
    src3 = src_p.reshape(NW, ROWS_PER_W, CHUNK)
    dst3 = dst_p.reshape(NW, ROWS_PER_W, CHUNK)

    parts = _sc_segment_sum(s, src3, dst3)

    return _tc_forward(
        data_x, parts, W_rel, b_rel.reshape(1, 1), W_root, W_hist,
        b_hist.reshape(1, HIDDEN), W_ls, b_ls.reshape(1, HIDDEN), W1,
        b1.reshape(1, HIDDEN), W2, b2.reshape(1, OUT_DIM))
